# Initial kernel scaffold; baseline (speedup 1.0000x reference)
import jax
import jax.numpy as jnp
from jax.experimental import pallas as pl


def kernel(x, table, lora_A, lora_B):
    raise NotImplementedError("write your pallas kernel here")


# speedup vs baseline: 7.3898x; 7.3898x over previous
import functools

import jax
import jax.numpy as jnp
from jax import lax
from jax.experimental import pallas as pl
from jax.experimental.pallas import tpu as pltpu
from jax.experimental.pallas import tpu_sc as plsc

D = 64
R = 16
IDX_W = 128
KI = 8
CHUNK = IDX_W * KI


def _make_sc_gather(n_tokens: int):
    info = plsc.get_sparse_core_info()
    nc, ns = info.num_cores, info.num_subcores
    nw = nc * ns
    per_w = n_tokens // nw
    assert n_tokens % (nw * CHUNK) == 0
    n_chunks = per_w // CHUNK
    rows_per_chunk = CHUNK // IDX_W

    mesh = plsc.VectorSubcoreMesh(core_axis_name="c", subcore_axis_name="s")

    @functools.partial(
        pl.kernel,
        mesh=mesh,
        compiler_params=pltpu.CompilerParams(use_tc_tiling_on_sc=False),
        out_type=[
            jax.ShapeDtypeStruct((n_tokens // IDX_W, IDX_W, D), jnp.float32),
            jax.ShapeDtypeStruct((n_tokens // IDX_W, IDX_W, R), jnp.float32),
        ],
        scratch_types=[
            pltpu.VMEM((KI, IDX_W), jnp.int32),
            pltpu.VMEM((KI, IDX_W, D), jnp.float32),
            pltpu.VMEM((KI, IDX_W, R), jnp.float32),
            pltpu.SemaphoreType.DMA,
            pltpu.SemaphoreType.DMA,
        ],
    )
    def sc_gather(idx_hbm, table_hbm, art_hbm, base_hbm, ar_hbm,
                  idx_v, rows_v, ar_v, sem_rows, sem_ar):
        wid = lax.axis_index("s") * nc + lax.axis_index("c")
        row0 = wid * (per_w // IDX_W)

        def body(i, _):
            rbase = row0 + i * rows_per_chunk
            pltpu.sync_copy(idx_hbm.at[pl.ds(rbase, rows_per_chunk)], idx_v)
            cps = []
            for j in range(KI):
                cps.append(pltpu.async_copy(
                    table_hbm.at[idx_v.at[j]], rows_v.at[j], sem_rows))
                cps.append(pltpu.async_copy(
                    art_hbm.at[idx_v.at[j]], ar_v.at[j], sem_ar))
            for cp in cps:
                cp.wait()
            pltpu.sync_copy(rows_v, base_hbm.at[pl.ds(rbase, rows_per_chunk)])
            pltpu.sync_copy(ar_v, ar_hbm.at[pl.ds(rbase, rows_per_chunk)])
            return 0

        lax.fori_loop(0, n_chunks, body, 0)

    return sc_gather


def _tc_combine_body(base_ref, ar_ref, lb_ref, out_ref):
    lora = lax.dot_general(
        ar_ref[...], lb_ref[...],
        dimension_numbers=(((1,), (1,)), ((), ())),
        preferred_element_type=jnp.float32)
    out_ref[...] = base_ref[...] + lora


def _tc_combine(base, ar, lora_B, bm: int):
    n = base.shape[0]
    assert n % bm == 0
    return pl.pallas_call(
        _tc_combine_body,
        grid=(n // bm,),
        in_specs=[
            pl.BlockSpec((bm, D), lambda i: (i, 0)),
            pl.BlockSpec((bm, R), lambda i: (i, 0)),
            pl.BlockSpec((D, R), lambda i: (0, 0)),
        ],
        out_specs=pl.BlockSpec((bm, D), lambda i: (i, 0)),
        out_shape=jax.ShapeDtypeStruct((n, D), jnp.float32),
    )(base, ar, lora_B)


def kernel(x, table, lora_A, lora_B):
    b, s = x.shape
    n = b * s
    idx = x.reshape(n // IDX_W, IDX_W)
    art = lora_A.T

    base3, ar3 = _make_sc_gather(n)(idx, table, art)
    base = base3.reshape(n, D)
    ar = ar3.reshape(n, R)
    out = _tc_combine(base, ar, lora_B, bm=4096)
    return out.reshape(b, s, D)
